# Initial kernel scaffold; baseline (speedup 1.0000x reference)
import functools

import jax
import jax.numpy as jnp
from jax import lax
from jax.experimental import pallas as pl
from jax.experimental.pallas import tpu as pltpu
from jax.experimental.pallas import tpu_sc as plsc

N = 10000
E = 320000
D = 128
B = 16
L = 2048

NC = 2
NS = 16
NW = NC * NS
EP = E // NW
CH = 80
NCHUNK = EP // CH
RPT = N // NS

NPAD = 10240
TQ = B * L
QP = TQ // NW
QCH = 128
QNCH = QP // QCH

_mesh = functools.partial(
    plsc.VectorSubcoreMesh,
    core_axis_name="c", subcore_axis_name="s", num_cores=NC, num_subcores=NS)


@functools.partial(
    pl.kernel,
    out_type=jax.ShapeDtypeStruct((NC, N, 16), jnp.float32),
    mesh=_mesh(),
    scratch_types=[
        pltpu.VMEM((NCHUNK, CH), jnp.int32),
        pltpu.VMEM((CH, 16), jnp.float32),
        pltpu.VMEM_SHARED((N, 16), jnp.float32),
    ],
)
def _deg_kernel(dst_hbm, ones_hbm, zeros_hbm, deg_out, dst_vm, ones_vm,
                deg_sp):
    c = lax.axis_index("c")
    s = lax.axis_index("s")
    pltpu.sync_copy(dst_hbm.at[c, s], dst_vm)
    pltpu.sync_copy(ones_hbm, ones_vm)
    base = s * RPT
    pltpu.sync_copy(zeros_hbm, deg_sp.at[pl.ds(base, RPT)])
    plsc.subcore_barrier()

    def body(j, carry):
        pltpu.sync_copy(ones_vm, deg_sp.at[dst_vm.at[j]], add=True)
        return carry

    lax.fori_loop(0, NCHUNK, body, 0)
    plsc.subcore_barrier()
    pltpu.sync_copy(deg_sp.at[pl.ds(base, RPT)],
                    deg_out.at[c, pl.ds(base, RPT)])


@functools.partial(
    pl.kernel,
    out_type=jax.ShapeDtypeStruct((NC, N, D), jnp.float32),
    mesh=_mesh(),
    scratch_types=[
        pltpu.VMEM((NCHUNK, CH), jnp.int32),
        pltpu.VMEM((NCHUNK, CH), jnp.int32),
        pltpu.VMEM((CH, D), jnp.float32),
        pltpu.VMEM_SHARED((N, D), jnp.float32),
    ],
)
def _msg_kernel(y_hbm, src_hbm, dst_hbm, zeros_hbm, s_out, src_vm, dst_vm,
                buf, s_sp):
    c = lax.axis_index("c")
    s = lax.axis_index("s")
    pltpu.sync_copy(src_hbm.at[c, s], src_vm)
    pltpu.sync_copy(dst_hbm.at[c, s], dst_vm)
    base = s * RPT
    pltpu.sync_copy(zeros_hbm, s_sp.at[pl.ds(base, RPT)])
    plsc.subcore_barrier()

    def body(j, carry):
        pltpu.sync_copy(y_hbm.at[src_vm.at[j]], buf)
        pltpu.sync_copy(buf, s_sp.at[dst_vm.at[j]], add=True)
        return carry

    lax.fori_loop(0, NCHUNK, body, 0)
    plsc.subcore_barrier()
    pltpu.sync_copy(s_sp.at[pl.ds(base, RPT)], s_out.at[c, pl.ds(base, RPT)])


@functools.partial(
    pl.kernel,
    out_type=jax.ShapeDtypeStruct((TQ, D), jnp.float32),
    mesh=_mesh(),
    scratch_types=[
        pltpu.VMEM((QNCH, QCH), jnp.int32),
        pltpu.VMEM((QCH, D), jnp.float32),
    ],
)
def _traj_kernel(road_hbm, idx_hbm, out_hbm, idx_vm, buf):
    c = lax.axis_index("c")
    s = lax.axis_index("s")
    wid = s * NC + c
    pltpu.sync_copy(idx_hbm.at[wid], idx_vm)
    obase = wid * QP

    def body(j, carry):
        pltpu.sync_copy(road_hbm.at[idx_vm.at[j]], buf)
        pltpu.sync_copy(buf, out_hbm.at[pl.ds(obase + j * QCH, QCH)])
        return carry

    lax.fori_loop(0, QNCH, body, 0)


_TC1_BLK = 2000


def _tc1_body(nf_ref, w_ref, dg_ref, y_ref):
    x = jnp.dot(nf_ref[...], w_ref[...], preferred_element_type=jnp.float32)
    deg = dg_ref[0, :, 0:1] + dg_ref[1, :, 0:1] + 1.0
    y_ref[...] = x * lax.rsqrt(deg)


def _tc1(node_feat, w, deg2):
    return pl.pallas_call(
        _tc1_body,
        grid=(N // _TC1_BLK,),
        in_specs=[
            pl.BlockSpec((_TC1_BLK, D), lambda i: (i, 0)),
            pl.BlockSpec((D, D), lambda i: (0, 0)),
            pl.BlockSpec((NC, _TC1_BLK, 16), lambda i: (0, i, 0)),
        ],
        out_specs=pl.BlockSpec((_TC1_BLK, D), lambda i: (i, 0)),
        out_shape=jax.ShapeDtypeStruct((N, D), jnp.float32),
    )(node_feat, w, deg2)


_TC2_BLK = 512


def _tc2_body(s_ref, dg_ref, y_ref, b_ref, traj_ref, sl_ref, road_ref,
              idxm_ref):
    i = pl.program_id(0)
    deg = dg_ref[0, :, 0:1] + dg_ref[1, :, 0:1] + 1.0
    dinv = lax.rsqrt(deg)
    acc = s_ref[0] + s_ref[1] + y_ref[...]
    val = jnp.maximum(dinv * acc + b_ref[...], 0.0)
    row = i * _TC2_BLK + lax.broadcasted_iota(jnp.int32, (_TC2_BLK, 1), 0)
    road_ref[...] = jnp.where(row < N, val, 0.0)
    pos = lax.broadcasted_iota(jnp.int32, (B, L), 1)
    idxm_ref[...] = jnp.where(pos < sl_ref[...], traj_ref[...], N)


def _tc2(s2, deg2, y, b, traj, seq_len):
    return pl.pallas_call(
        _tc2_body,
        grid=(NPAD // _TC2_BLK,),
        in_specs=[
            pl.BlockSpec((NC, _TC2_BLK, D), lambda i: (0, i, 0)),
            pl.BlockSpec((NC, _TC2_BLK, 16), lambda i: (0, i, 0)),
            pl.BlockSpec((_TC2_BLK, D), lambda i: (i, 0)),
            pl.BlockSpec((1, D), lambda i: (0, 0)),
            pl.BlockSpec((B, L), lambda i: (0, 0)),
            pl.BlockSpec((B, 1), lambda i: (0, 0)),
        ],
        out_specs=[
            pl.BlockSpec((_TC2_BLK, D), lambda i: (i, 0)),
            pl.BlockSpec((B, L), lambda i: (0, 0)),
        ],
        out_shape=[
            jax.ShapeDtypeStruct((NPAD, D), jnp.float32),
            jax.ShapeDtypeStruct((B, L), jnp.int32),
        ],
    )(s2, deg2, y, b.reshape(1, D), traj, seq_len.reshape(B, 1))


def kernel(traj_seqs, seq_len, node_feat, edge_index, W, b):
    src = edge_index[0].astype(jnp.int32).reshape(NC, NS, NCHUNK, CH)
    dst = edge_index[1].astype(jnp.int32).reshape(NC, NS, NCHUNK, CH)
    ones16 = jnp.ones((CH, 16), jnp.float32)
    zeros16 = jnp.zeros((RPT, 16), jnp.float32)
    zerosd = jnp.zeros((RPT, D), jnp.float32)

    deg2 = _deg_kernel(dst, ones16, zeros16)
    y = _tc1(node_feat, W, deg2)
    s2 = _msg_kernel(y, src, dst, zerosd)
    road, idxm = _tc2(s2, deg2, y, b, traj_seqs[..., 0], seq_len)
    out = _traj_kernel(road, idxm.reshape(NW, QNCH, QCH))
    return out.reshape(B, L, D)


# speedup vs baseline: 5.2126x; 5.2126x over previous
import functools

import jax
import jax.numpy as jnp
from jax import lax
from jax.experimental import pallas as pl
from jax.experimental.pallas import tpu as pltpu
from jax.experimental.pallas import tpu_sc as plsc

N = 10000
E = 320000
D = 128
B = 16
L = 2048

NC = 2
NS = 16
NW = NC * NS
CH = 128
NCHUNK = 80
EPW = NCHUNK * CH
EPAD = NW * EPW
NPAD = 10240
PADN = NPAD - 8
RPT = NPAD // NS

TQ = B * L
QP = TQ // NW
QCH = 128
QNCH = QP // QCH

_mesh = functools.partial(
    plsc.VectorSubcoreMesh,
    core_axis_name="c", subcore_axis_name="s", num_cores=NC, num_subcores=NS)


@functools.partial(
    pl.kernel,
    out_type=jax.ShapeDtypeStruct((NC, NPAD, D), jnp.float32),
    mesh=_mesh(),
    scratch_types=[
        pltpu.VMEM((NCHUNK, CH), jnp.int32),
        pltpu.VMEM((CH, D), jnp.float32),
        pltpu.VMEM_SHARED((NPAD, D), jnp.float32),
    ],
)
def _deg_kernel(dst_hbm, ones_hbm, zeros_hbm, deg_out, dst_vm, ones_vm,
                deg_sp):
    c = lax.axis_index("c")
    s = lax.axis_index("s")
    pltpu.sync_copy(dst_hbm.at[c, s], dst_vm)
    pltpu.sync_copy(ones_hbm, ones_vm)
    base = s * RPT
    pltpu.sync_copy(zeros_hbm, deg_sp.at[pl.ds(base, RPT)])
    plsc.subcore_barrier()

    def body(j, carry):
        pltpu.sync_copy(ones_vm, deg_sp.at[dst_vm.at[j]], add=True)
        return carry

    lax.fori_loop(0, NCHUNK, body, 0)
    plsc.subcore_barrier()
    pltpu.sync_copy(deg_sp.at[pl.ds(base, RPT)],
                    deg_out.at[c, pl.ds(base, RPT)])


@functools.partial(
    pl.kernel,
    out_type=jax.ShapeDtypeStruct((NC, NPAD, D), jnp.float32),
    mesh=_mesh(),
    scratch_types=[
        pltpu.VMEM((NCHUNK, CH), jnp.int32),
        pltpu.VMEM((NCHUNK, CH), jnp.int32),
        pltpu.VMEM((CH, D), jnp.float32),
        pltpu.VMEM_SHARED((NPAD, D), jnp.float32),
    ],
)
def _msg_kernel(y_hbm, src_hbm, dst_hbm, zeros_hbm, s_out, src_vm, dst_vm,
                buf, s_sp):
    c = lax.axis_index("c")
    s = lax.axis_index("s")
    pltpu.sync_copy(src_hbm.at[c, s], src_vm)
    pltpu.sync_copy(dst_hbm.at[c, s], dst_vm)
    base = s * RPT
    pltpu.sync_copy(zeros_hbm, s_sp.at[pl.ds(base, RPT)])
    plsc.subcore_barrier()

    def body(j, carry):
        pltpu.sync_copy(y_hbm.at[src_vm.at[j]], buf)
        pltpu.sync_copy(buf, s_sp.at[dst_vm.at[j]], add=True)
        return carry

    lax.fori_loop(0, NCHUNK, body, 0)
    plsc.subcore_barrier()
    pltpu.sync_copy(s_sp.at[pl.ds(base, RPT)], s_out.at[c, pl.ds(base, RPT)])


@functools.partial(
    pl.kernel,
    out_type=jax.ShapeDtypeStruct((TQ, D), jnp.float32),
    mesh=_mesh(),
    scratch_types=[
        pltpu.VMEM((QNCH, QCH), jnp.int32),
        pltpu.VMEM((QCH, D), jnp.float32),
    ],
)
def _traj_kernel(road_hbm, idx_hbm, out_hbm, idx_vm, buf):
    c = lax.axis_index("c")
    s = lax.axis_index("s")
    wid = s * NC + c
    pltpu.sync_copy(idx_hbm.at[wid], idx_vm)
    obase = wid * QP

    def body(j, carry):
        pltpu.sync_copy(road_hbm.at[idx_vm.at[j]], buf)
        pltpu.sync_copy(buf, out_hbm.at[pl.ds(obase + j * QCH, QCH)])
        return carry

    lax.fori_loop(0, QNCH, body, 0)


_TC1_BLK = 2048


def _tc1_body(nf_ref, w_ref, dg_ref, y_ref):
    x = jnp.dot(nf_ref[...], w_ref[...], preferred_element_type=jnp.float32)
    deg = dg_ref[0, :, 0:1] + dg_ref[1, :, 0:1] + 1.0
    y_ref[...] = x * lax.rsqrt(deg)


def _tc1(node_feat, w, deg2):
    return pl.pallas_call(
        _tc1_body,
        grid=(NPAD // _TC1_BLK,),
        in_specs=[
            pl.BlockSpec((_TC1_BLK, D), lambda i: (i, 0)),
            pl.BlockSpec((D, D), lambda i: (0, 0)),
            pl.BlockSpec((NC, _TC1_BLK, D), lambda i: (0, i, 0)),
        ],
        out_specs=pl.BlockSpec((_TC1_BLK, D), lambda i: (i, 0)),
        out_shape=jax.ShapeDtypeStruct((NPAD, D), jnp.float32),
    )(node_feat, w, deg2)


_TC2_BLK = 512


def _tc2_body(s_ref, dg_ref, y_ref, b_ref, traj_ref, sl_ref, road_ref,
              idxm_ref):
    i = pl.program_id(0)
    deg = dg_ref[0, :, 0:1] + dg_ref[1, :, 0:1] + 1.0
    dinv = lax.rsqrt(deg)
    acc = s_ref[0] + s_ref[1] + y_ref[...]
    val = jnp.maximum(dinv * acc + b_ref[...], 0.0)
    row = i * _TC2_BLK + lax.broadcasted_iota(jnp.int32, (_TC2_BLK, 1), 0)
    road_ref[...] = jnp.where(row < N, val, 0.0)
    pos = lax.broadcasted_iota(jnp.int32, (B, L), 1)
    idxm_ref[...] = jnp.where(pos < sl_ref[...], traj_ref[...], N)


def _tc2(s2, deg2, y, b, traj, seq_len):
    return pl.pallas_call(
        _tc2_body,
        grid=(NPAD // _TC2_BLK,),
        in_specs=[
            pl.BlockSpec((NC, _TC2_BLK, D), lambda i: (0, i, 0)),
            pl.BlockSpec((NC, _TC2_BLK, D), lambda i: (0, i, 0)),
            pl.BlockSpec((_TC2_BLK, D), lambda i: (i, 0)),
            pl.BlockSpec((1, D), lambda i: (0, 0)),
            pl.BlockSpec((B, L), lambda i: (0, 0)),
            pl.BlockSpec((B, 1), lambda i: (0, 0)),
        ],
        out_specs=[
            pl.BlockSpec((_TC2_BLK, D), lambda i: (i, 0)),
            pl.BlockSpec((B, L), lambda i: (0, 0)),
        ],
        out_shape=[
            jax.ShapeDtypeStruct((NPAD, D), jnp.float32),
            jax.ShapeDtypeStruct((B, L), jnp.int32),
        ],
    )(s2, deg2, y, b.reshape(1, D), traj, seq_len.reshape(B, 1))


def _pad_edges(idx):
    pad = jnp.full((EPAD - E,), PADN, dtype=jnp.int32)
    return jnp.concatenate([idx.astype(jnp.int32), pad]).reshape(
        NC, NS, NCHUNK, CH)


def kernel(traj_seqs, seq_len, node_feat, edge_index, W, b):
    src = _pad_edges(edge_index[0])
    dst = _pad_edges(edge_index[1])
    onesd = jnp.ones((CH, D), jnp.float32)
    zerosd = jnp.zeros((RPT, D), jnp.float32)

    deg2 = _deg_kernel(dst, onesd, zerosd)
    y = _tc1(node_feat, W, deg2)
    s2 = _msg_kernel(y, src, dst, zerosd)
    road, idxm = _tc2(s2, deg2, y, b, traj_seqs[..., 0], seq_len)
    out = _traj_kernel(road, idxm.reshape(NW, QNCH, QCH))
    return out.reshape(B, L, D)


# speedup vs baseline: 5.4437x; 1.0443x over previous
import functools

import jax
import jax.numpy as jnp
from jax import lax
from jax.experimental import pallas as pl
from jax.experimental.pallas import tpu as pltpu
from jax.experimental.pallas import tpu_sc as plsc

N = 10000
E = 320000
D = 128
B = 16
L = 2048

NC = 2
NS = 16
NW = NC * NS
CH = 128
NCHUNK = 80
EPW = NCHUNK * CH
EPAD = NW * EPW
NPAD = 10240
PADN = NPAD - 8
RPT = NPAD // NS

TQ = B * L
QP = TQ // NW
QCH = 128
QNCH = QP // QCH

_mesh = functools.partial(
    plsc.VectorSubcoreMesh,
    core_axis_name="c", subcore_axis_name="s", num_cores=NC, num_subcores=NS)


_DEG_GRP = 8


@functools.partial(
    pl.kernel,
    out_type=jax.ShapeDtypeStruct((NC, NPAD, D), jnp.float32),
    mesh=_mesh(),
    scratch_types=[
        pltpu.VMEM((NCHUNK, CH), jnp.int32),
        pltpu.VMEM((CH, D), jnp.float32),
        pltpu.SemaphoreType.DMA,
        pltpu.VMEM_SHARED((NPAD, D), jnp.float32),
    ],
)
def _deg_kernel(dst_hbm, ones_hbm, zeros_hbm, deg_out, dst_vm, ones_vm,
                sem, deg_sp):
    c = lax.axis_index("c")
    s = lax.axis_index("s")
    pltpu.sync_copy(dst_hbm.at[c, s], dst_vm)
    pltpu.sync_copy(ones_hbm, ones_vm)
    base = s * RPT
    pltpu.sync_copy(zeros_hbm, deg_sp.at[pl.ds(base, RPT)])
    plsc.subcore_barrier()

    def body(g, carry):
        for k in range(_DEG_GRP):
            pltpu.async_copy(ones_vm, deg_sp.at[dst_vm.at[g * _DEG_GRP + k]],
                             sem, add=True)
        for k in range(_DEG_GRP):
            pltpu.make_async_copy(
                ones_vm, deg_sp.at[dst_vm.at[g * _DEG_GRP + k]], sem).wait()
        return carry

    lax.fori_loop(0, NCHUNK // _DEG_GRP, body, 0)
    plsc.subcore_barrier()
    pltpu.sync_copy(deg_sp.at[pl.ds(base, RPT)],
                    deg_out.at[c, pl.ds(base, RPT)])


@functools.partial(
    pl.kernel,
    out_type=jax.ShapeDtypeStruct((NC, NPAD, D), jnp.float32),
    mesh=_mesh(),
    scratch_types=[
        pltpu.VMEM((NCHUNK // 2, CH), jnp.int32),
        pltpu.VMEM((NCHUNK // 2, CH), jnp.int32),
        pltpu.VMEM((CH, D), jnp.float32),
        pltpu.VMEM((CH, D), jnp.float32),
        pltpu.SemaphoreType.DMA,
        pltpu.SemaphoreType.DMA,
        pltpu.SemaphoreType.DMA,
        pltpu.SemaphoreType.DMA,
        pltpu.VMEM_SHARED((NPAD, D), jnp.float32),
    ],
)
def _msg_kernel(y_hbm, src_hbm, dst_hbm, zeros_hbm, s_out, src_vm, dst_vm,
                buf_a, buf_b, gsem_a, gsem_b, ssem_a, ssem_b, s_sp):
    c = lax.axis_index("c")
    s = lax.axis_index("s")
    base = s * RPT
    pltpu.sync_copy(zeros_hbm, s_sp.at[pl.ds(base, RPT)])
    plsc.subcore_barrier()

    bufs = (buf_a, buf_b)
    gsems = (gsem_a, gsem_b)
    ssems = (ssem_a, ssem_b)
    half = NCHUNK // 2

    for p in range(2):
        pltpu.sync_copy(src_hbm.at[c, s, pl.ds(p * half, half)], src_vm)
        pltpu.sync_copy(dst_hbm.at[c, s, pl.ds(p * half, half)], dst_vm)
        pltpu.async_copy(y_hbm.at[src_vm.at[0]], buf_a, gsem_a)
        pltpu.async_copy(y_hbm.at[src_vm.at[1]], buf_b, gsem_b)

        def body(g, carry):
            for k in range(2):
                j = g * 2 + k
                buf, gsem, ssem = bufs[k], gsems[k], ssems[k]
                pltpu.make_async_copy(y_hbm.at[src_vm.at[j]], buf,
                                      gsem).wait()
                pltpu.async_copy(buf, s_sp.at[dst_vm.at[j]], ssem, add=True)

                @pl.when(j + 2 < half)
                def _():
                    pltpu.make_async_copy(buf, s_sp.at[dst_vm.at[j]],
                                          ssem).wait()
                    pltpu.async_copy(y_hbm.at[src_vm.at[j + 2]], buf, gsem)
            return carry

        lax.fori_loop(0, half // 2, body, 0)
        pltpu.make_async_copy(buf_a, s_sp.at[dst_vm.at[half - 2]],
                              ssem_a).wait()
        pltpu.make_async_copy(buf_b, s_sp.at[dst_vm.at[half - 1]],
                              ssem_b).wait()

    plsc.subcore_barrier()
    pltpu.sync_copy(s_sp.at[pl.ds(base, RPT)], s_out.at[c, pl.ds(base, RPT)])


@functools.partial(
    pl.kernel,
    out_type=jax.ShapeDtypeStruct((TQ, D), jnp.float32),
    mesh=_mesh(),
    scratch_types=[
        pltpu.VMEM((QNCH, QCH), jnp.int32),
        pltpu.VMEM((QCH, D), jnp.float32),
        pltpu.VMEM((QCH, D), jnp.float32),
        pltpu.SemaphoreType.DMA,
        pltpu.SemaphoreType.DMA,
        pltpu.SemaphoreType.DMA,
        pltpu.SemaphoreType.DMA,
    ],
)
def _traj_kernel(road_hbm, idx_hbm, out_hbm, idx_vm, buf_a, buf_b,
                 gsem_a, gsem_b, osem_a, osem_b):
    c = lax.axis_index("c")
    s = lax.axis_index("s")
    wid = s * NC + c
    pltpu.sync_copy(idx_hbm.at[wid], idx_vm)
    obase = wid * QP

    bufs = (buf_a, buf_b)
    gsems = (gsem_a, gsem_b)
    osems = (osem_a, osem_b)
    pltpu.async_copy(road_hbm.at[idx_vm.at[0]], buf_a, gsem_a)
    pltpu.async_copy(road_hbm.at[idx_vm.at[1]], buf_b, gsem_b)

    def body(g, carry):
        for k in range(2):
            j = g * 2 + k
            buf, gsem, osem = bufs[k], gsems[k], osems[k]
            dst = out_hbm.at[pl.ds(obase + j * QCH, QCH)]
            pltpu.make_async_copy(road_hbm.at[idx_vm.at[j]], buf, gsem).wait()
            pltpu.async_copy(buf, dst, osem)

            @pl.when(j + 2 < QNCH)
            def _():
                pltpu.make_async_copy(buf, dst, osem).wait()
                pltpu.async_copy(road_hbm.at[idx_vm.at[j + 2]], buf, gsem)
        return carry

    lax.fori_loop(0, QNCH // 2, body, 0)
    pltpu.make_async_copy(
        buf_a, out_hbm.at[pl.ds(obase + (QNCH - 2) * QCH, QCH)], osem_a).wait()
    pltpu.make_async_copy(
        buf_b, out_hbm.at[pl.ds(obase + (QNCH - 1) * QCH, QCH)], osem_b).wait()


_TC1_BLK = 2048


def _tc1_body(nf_ref, w_ref, dg_ref, y_ref):
    x = jnp.dot(nf_ref[...], w_ref[...], preferred_element_type=jnp.float32)
    deg = dg_ref[0, :, 0:1] + dg_ref[1, :, 0:1] + 1.0
    y_ref[...] = x * lax.rsqrt(deg)


def _tc1(node_feat, w, deg2):
    return pl.pallas_call(
        _tc1_body,
        grid=(NPAD // _TC1_BLK,),
        in_specs=[
            pl.BlockSpec((_TC1_BLK, D), lambda i: (i, 0)),
            pl.BlockSpec((D, D), lambda i: (0, 0)),
            pl.BlockSpec((NC, _TC1_BLK, D), lambda i: (0, i, 0)),
        ],
        out_specs=pl.BlockSpec((_TC1_BLK, D), lambda i: (i, 0)),
        out_shape=jax.ShapeDtypeStruct((NPAD, D), jnp.float32),
    )(node_feat, w, deg2)


_TC2_BLK = 512


def _tc2_body(s_ref, dg_ref, y_ref, b_ref, traj_ref, sl_ref, road_ref,
              idxm_ref):
    i = pl.program_id(0)
    deg = dg_ref[0, :, 0:1] + dg_ref[1, :, 0:1] + 1.0
    dinv = lax.rsqrt(deg)
    acc = s_ref[0] + s_ref[1] + y_ref[...]
    val = jnp.maximum(dinv * acc + b_ref[...], 0.0)
    row = i * _TC2_BLK + lax.broadcasted_iota(jnp.int32, (_TC2_BLK, 1), 0)
    road_ref[...] = jnp.where(row < N, val, 0.0)
    pos = lax.broadcasted_iota(jnp.int32, (B, L), 1)
    idxm_ref[...] = jnp.where(pos < sl_ref[...], traj_ref[...], N)


def _tc2(s2, deg2, y, b, traj, seq_len):
    return pl.pallas_call(
        _tc2_body,
        grid=(NPAD // _TC2_BLK,),
        in_specs=[
            pl.BlockSpec((NC, _TC2_BLK, D), lambda i: (0, i, 0)),
            pl.BlockSpec((NC, _TC2_BLK, D), lambda i: (0, i, 0)),
            pl.BlockSpec((_TC2_BLK, D), lambda i: (i, 0)),
            pl.BlockSpec((1, D), lambda i: (0, 0)),
            pl.BlockSpec((B, L), lambda i: (0, 0)),
            pl.BlockSpec((B, 1), lambda i: (0, 0)),
        ],
        out_specs=[
            pl.BlockSpec((_TC2_BLK, D), lambda i: (i, 0)),
            pl.BlockSpec((B, L), lambda i: (0, 0)),
        ],
        out_shape=[
            jax.ShapeDtypeStruct((NPAD, D), jnp.float32),
            jax.ShapeDtypeStruct((B, L), jnp.int32),
        ],
    )(s2, deg2, y, b.reshape(1, D), traj, seq_len.reshape(B, 1))


def _pad_edges(idx):
    pad = jnp.full((EPAD - E,), PADN, dtype=jnp.int32)
    return jnp.concatenate([idx.astype(jnp.int32), pad]).reshape(
        NC, NS, NCHUNK, CH)


def kernel(traj_seqs, seq_len, node_feat, edge_index, W, b):
    src = _pad_edges(edge_index[0])
    dst = _pad_edges(edge_index[1])
    onesd = jnp.ones((CH, D), jnp.float32)
    zerosd = jnp.zeros((RPT, D), jnp.float32)

    deg2 = _deg_kernel(dst, onesd, zerosd)
    y = _tc1(node_feat, W, deg2)
    s2 = _msg_kernel(y, src, dst, zerosd)
    road, idxm = _tc2(s2, deg2, y, b, traj_seqs[..., 0], seq_len)
    out = _traj_kernel(road, idxm.reshape(NW, QNCH, QCH))
    return out.reshape(B, L, D)
